# baseline (speedup 1.0000x reference)
import functools

import jax
import jax.numpy as jnp
from jax.experimental import pallas as pl
from jax.experimental.pallas import tpu as pltpu

_VMEM = 64 * 1024 * 1024


def _act(y, kind):
    if kind == "relu":
        return jnp.maximum(y, 0.0)
    if kind == "elu":
        return jnp.where(y > 0, y, jnp.exp(jnp.minimum(y, 0.0)) - 1.0)
    return y


def _dotT(x, w):
    return jax.lax.dot_general(x, w, (((1,), (1,)), ((), ())),
                               preferred_element_type=jnp.float32)


def _pad_hw(y3):
    h, w, c = y3.shape
    zr = jnp.zeros((1, w, c), y3.dtype)
    y3 = jnp.concatenate([zr, y3, zr], axis=0)
    zc = jnp.zeros((h + 2, 1, c), y3.dtype)
    return jnp.concatenate([zc, y3, zc], axis=1)


def _phase_split(yp):
    H, W, c = yp.shape
    return (yp.reshape(H // 2, 2, W // 2, 2, c)
            .transpose(1, 3, 0, 2, 4))


def _tap_cat(xp, oh, ow):
    c = xp.shape[-1]
    parts = [xp[kh:kh + oh, kw:kw + ow, :].reshape(oh * ow, c)
             .astype(jnp.bfloat16)
             for kh in range(3) for kw in range(3)]
    return jnp.concatenate(parts, axis=1)


def _tap_cat_s2(ph, oh, ow):
    c = ph.shape[-1]
    parts = []
    for kh in range(3):
        for kw in range(3):
            xs = ph[kh % 2, kw % 2,
                    kh // 2:kh // 2 + oh, kw // 2:kw // 2 + ow, :]
            parts.append(xs.reshape(oh * ow, c).astype(jnp.bfloat16))
    return jnp.concatenate(parts, axis=1)


def _front_body(x_ref, w0_ref, b0_ref, w1_ref, b1_ref, w2_ref, b2_ref,
                o_ref, *, T):
    t = pl.program_id(1)
    top = x_ref[0, pl.ds(t + 1, 1)][0]
    bot = x_ref[0, pl.ds(t, 1)][0]
    d4 = top - bot
    t9 = jnp.stack([d4[kh % 2, kw % 2,
                       kh // 2:kh // 2 + 64, kw // 2:kw // 2 + 64]
                    for kh in range(3) for kw in range(3)], axis=-1)
    a0 = t9.reshape(4096, 9).astype(jnp.bfloat16)
    y0 = jnp.dot(a0, w0_ref[...], preferred_element_type=jnp.float32)
    y0 = _act(y0 + b0_ref[...], "elu")
    a1 = _tap_cat(_pad_hw(y0.reshape(64, 64, 64)), 64, 64)
    y1 = jnp.dot(a1, w1_ref[...], preferred_element_type=jnp.float32)
    y1 = _act(y1 + b1_ref[...], "elu")
    p1 = _phase_split(_pad_hw(y1.reshape(64, 64, 64)))
    a2 = _tap_cat_s2(p1, 32, 32)
    y2 = jnp.dot(a2, w2_ref[...], preferred_element_type=jnp.float32)
    y2 = _act(y2 + b2_ref[...], "elu")
    o_ref[...] = _pad_hw(y2.reshape(32, 32, 128))[None]


def _tail_body(x_ref, w3_ref, b3_ref, w4_ref, b4_ref, w5_ref, b5_ref,
               o0_ref, o1_ref, o2_ref):
    a3 = _tap_cat(x_ref[0], 32, 32)
    y3 = jnp.dot(a3, w3_ref[...], preferred_element_type=jnp.float32)
    y3 = _act(y3 + b3_ref[...], "elu")
    o0_ref[...] = y3.astype(jnp.bfloat16)
    p3 = _phase_split(_pad_hw(y3.reshape(32, 32, 128)))
    a4 = _tap_cat_s2(p3, 16, 16)
    y4 = jnp.dot(a4, w4_ref[...], preferred_element_type=jnp.float32)
    y4 = jnp.maximum(y4 + b4_ref[...], 0.0)
    o1_ref[...] = y4.astype(jnp.bfloat16)
    p4 = _phase_split(_pad_hw(y4.reshape(16, 16, 256)))
    a5 = _tap_cat_s2(p4, 8, 8)
    y5 = jnp.dot(a5, w5_ref[...], preferred_element_type=jnp.float32)
    o2_ref[...] = jnp.maximum(y5 + b5_ref[...], 0.0).astype(jnp.bfloat16)


def _wcat(w):
    OC, IC = w.shape[0], w.shape[1]
    return (w.transpose(2, 3, 1, 0).reshape(9 * IC, OC)
            .astype(jnp.bfloat16))


def _read(q, mem_n, mem_raw):
    qn = q * jax.lax.rsqrt(
        jnp.maximum(jnp.sum(q * q, axis=1, keepdims=True), 1e-24))
    s = _dotT(qn.astype(jnp.bfloat16), mem_n)
    s = jnp.exp(s - jnp.max(s, axis=1, keepdims=True))
    p = s / jnp.sum(s, axis=1, keepdims=True)
    return jnp.dot(p.astype(jnp.bfloat16), mem_raw,
                   preferred_element_type=jnp.float32)


def _patch4(x4d):
    b, h, w, C = x4d.shape
    xp = jnp.concatenate([x4d, jnp.zeros((b, 1, w, C), x4d.dtype)], axis=1)
    xp = jnp.concatenate([xp, jnp.zeros((b, h + 1, 1, C), x4d.dtype)], axis=2)
    cols = jnp.concatenate(
        [xp[:, di:di + h, dj:dj + w, :] for di in (0, 1) for dj in (0, 1)],
        axis=-1)
    return cols.reshape(b * h * w, 4 * C)


def _shuffle_rows(y, b, h, w, oc):
    return (y.reshape(b, h, w, 2, 2, oc).transpose(0, 1, 3, 2, 4, 5)
            .reshape(b, 2 * h, 2 * w, oc))


def _tile4(bias_ref):
    bv = bias_ref[...]
    return jnp.concatenate([bv, bv, bv, bv], axis=1)


def _finish_mlp(acc, b1_ref, w2_ref, b2_ref):
    h = jnp.maximum(acc + b1_ref[...], 0.0).astype(jnp.bfloat16)
    return jnp.maximum(_dotT(h, w2_ref[...]) + b2_ref[...], 0.0)


def _head2_body(x_ref, w1_ref, b1_ref, w2_ref, b2_ref, mn_ref, mr_ref,
                wc_ref, bc_ref, o_ref, acc_ref, *, bh, T):
    t = pl.program_id(1)

    @pl.when(t == 0)
    def _():
        acc_ref[...] = jnp.zeros_like(acc_ref)

    xr = x_ref[...].reshape(bh * 64, 512)
    acc_ref[...] += _dotT(xr, w1_ref[...])

    @pl.when(t == T - 1)
    def _():
        r2 = _finish_mlp(acc_ref[...], b1_ref, w2_ref, b2_ref)
        mf = _read(r2, mn_ref[...], mr_ref[...])
        cols = _patch4(mf.astype(jnp.bfloat16).reshape(bh, 8, 8, 512))
        y = jnp.dot(cols, wc_ref[...], preferred_element_type=jnp.float32)
        o_ref[...] = jnp.maximum(y + _tile4(bc_ref), 0.0).astype(o_ref.dtype)


def _head1_body(x_ref, y2_ref, w1_ref, b1_ref, w2_ref, b2_ref, mn_ref,
                mr_ref, w10_ref, b10_ref, w11_ref, b11_ref,
                o10_ref, o11_ref, acc_ref, *, bh, T):
    t = pl.program_id(1)

    @pl.when(t == 0)
    def _():
        acc_ref[...] = jnp.zeros_like(acc_ref)

    xr = x_ref[...].reshape(bh * 256, 256)
    acc_ref[...] += _dotT(xr, w1_ref[...])

    @pl.when(t == T - 1)
    def _():
        r1 = _finish_mlp(acc_ref[...], b1_ref, w2_ref, b2_ref)
        mf2 = _shuffle_rows(y2_ref[...], bh, 8, 8, 256)
        mq = (r1.astype(jnp.bfloat16)
              + mf2.reshape(bh * 256, 256)).astype(jnp.float32)
        mf = _read(mq, mn_ref[...], mr_ref[...])
        cols = _patch4(mf.astype(jnp.bfloat16).reshape(bh, 16, 16, 256))
        y = jnp.dot(cols, w10_ref[...], preferred_element_type=jnp.float32)
        o10_ref[...] = jnp.maximum(y + _tile4(b10_ref), 0.0
                                   ).astype(o10_ref.dtype)
        cols2 = _patch4(mf2)
        y2 = jnp.dot(cols2, w11_ref[...], preferred_element_type=jnp.float32)
        o11_ref[...] = jnp.maximum(y2 + _tile4(b11_ref), 0.0
                                   ).astype(o11_ref.dtype)


def _head0_body(x_ref, y10_ref, y11_ref, w1_ref, b1_ref, w2_ref, b2_ref,
                mn_ref, mr_ref, wf_ref, bf_ref, o_ref, acc_ref, *, bh, T):
    t = pl.program_id(1)

    @pl.when(t == 0)
    def _():
        acc_ref[...] = jnp.zeros_like(acc_ref)

    xr = x_ref[...].reshape(bh * 1024, 128)
    acc_ref[...] += _dotT(xr, w1_ref[...])

    @pl.when(t == T - 1)
    def _():
        r0 = _finish_mlp(acc_ref[...], b1_ref, w2_ref, b2_ref)
        mf1 = _shuffle_rows(y10_ref[...].reshape(bh * 256, 512),
                            bh, 16, 16, 128).reshape(bh * 1024, 128)
        up2 = _shuffle_rows(y11_ref[...].reshape(bh * 256, 512),
                            bh, 16, 16, 128).reshape(bh * 1024, 128)
        mq = (r0.astype(jnp.bfloat16) + mf1).astype(jnp.float32)
        mf0 = _read(mq, mn_ref[...], mr_ref[...])
        wf = wf_ref[...]
        f32 = jnp.float32
        y = (jnp.dot(mf0.astype(jnp.bfloat16), wf[:128],
                     preferred_element_type=f32)
             + jnp.dot(mf1, wf[128:256], preferred_element_type=f32)
             + jnp.dot(up2, wf[256:], preferred_element_type=f32))
        y = jnp.maximum(y + bf_ref[...], 0.0)
        yt = jnp.transpose(y.reshape(bh, 1024, 128), (0, 2, 1))
        o_ref[...] = yt.reshape(bh, 128, 32, 32)


def _norm_rows(m):
    mf = m.astype(jnp.float32)
    return (mf * jax.lax.rsqrt(
        jnp.maximum(jnp.sum(mf * mf, axis=1, keepdims=True), 1e-24))
            ).astype(jnp.bfloat16)


def _convT_weight(w):
    IC, OC = w.shape[0], w.shape[1]
    wp = jnp.pad(w, ((0, 0), (0, 0), (1, 1), (1, 1)))
    blocks = jnp.stack(
        [wp[:, :, 2 - 2 * di:4 - 2 * di, 2 - 2 * dj:4 - 2 * dj]
         for di in (0, 1) for dj in (0, 1)], axis=0)
    wc = blocks.transpose(0, 1, 3, 4, 2).reshape(4 * IC, 4 * OC)
    return wc.astype(jnp.bfloat16)


def _bspec(shape):
    n = len(shape)
    return pl.BlockSpec(shape, lambda i, t, _n=n: (0,) * _n)


def kernel(enc_se0_w, enc_se0_b, enc_se1_w, enc_se1_b, enc_se2_w, enc_se2_b,
           enc_se3_w, enc_se3_b, enc_sc01_w, enc_sc01_b, enc_sc12_w,
           enc_sc12_b, enc_t001_w, enc_t001_b, enc_t012_w, enc_t012_b,
           enc_t101_w, enc_t101_b, enc_t112_w, enc_t112_b, enc_t201_w,
           enc_t201_b, enc_t212_w, enc_t212_b, emb21_w, emb21_b, emb11_w,
           emb11_b, emb10_w, emb10_b, fusion_w, fusion_b, memory_w0,
           memory_w1, memory_w2, memory_x):
    B, L = memory_x.shape[0], memory_x.shape[1]
    T = L - 1
    H = memory_x.shape[3]
    N = B * T

    xpad = jnp.pad(memory_x[:, :, 0], ((0, 0), (0, 0), (1, 1), (1, 1)))
    Hh = (H + 2) // 2
    xph = (xpad.reshape(B, L, Hh, 2, Hh, 2)
           .transpose(0, 1, 3, 5, 2, 4))
    w0 = enc_se0_w.reshape(64, 9).T.astype(jnp.bfloat16)
    xp3 = pl.pallas_call(
        functools.partial(_front_body, T=T),
        out_shape=jax.ShapeDtypeStruct((N, 34, 34, 128), jnp.float32),
        grid=(B, T),
        in_specs=[
            pl.BlockSpec((1, L, 2, 2, Hh, Hh),
                         lambda b, t: (b, 0, 0, 0, 0, 0)),
            pl.BlockSpec((9, 64), lambda b, t: (0, 0)),
            pl.BlockSpec((1, 64), lambda b, t: (0, 0)),
            pl.BlockSpec((576, 64), lambda b, t: (0, 0)),
            pl.BlockSpec((1, 64), lambda b, t: (0, 0)),
            pl.BlockSpec((576, 128), lambda b, t: (0, 0)),
            pl.BlockSpec((1, 128), lambda b, t: (0, 0)),
        ],
        out_specs=pl.BlockSpec((1, 34, 34, 128),
                               lambda b, t, _T=T: (b * _T + t, 0, 0, 0)),
        compiler_params=pltpu.CompilerParams(
            dimension_semantics=("parallel", "parallel"),
            vmem_limit_bytes=_VMEM),
    )(xph, w0, enc_se0_b.reshape(1, 64).astype(jnp.float32),
      _wcat(enc_se1_w), enc_se1_b.reshape(1, 64).astype(jnp.float32),
      _wcat(enc_se2_w), enc_se2_b.reshape(1, 128).astype(jnp.float32))

    x0r, x1r, x2r = pl.pallas_call(
        _tail_body,
        out_shape=(jax.ShapeDtypeStruct((N * 1024, 128), jnp.bfloat16),
                   jax.ShapeDtypeStruct((N * 256, 256), jnp.bfloat16),
                   jax.ShapeDtypeStruct((N * 64, 512), jnp.bfloat16)),
        grid=(N,),
        in_specs=[
            pl.BlockSpec((1, 34, 34, 128), lambda i: (i, 0, 0, 0)),
            pl.BlockSpec((1152, 128), lambda i: (0, 0)),
            pl.BlockSpec((1, 128), lambda i: (0, 0)),
            pl.BlockSpec((1152, 256), lambda i: (0, 0)),
            pl.BlockSpec((1, 256), lambda i: (0, 0)),
            pl.BlockSpec((2304, 512), lambda i: (0, 0)),
            pl.BlockSpec((1, 512), lambda i: (0, 0)),
        ],
        out_specs=(pl.BlockSpec((1024, 128), lambda i: (i, 0)),
                   pl.BlockSpec((256, 256), lambda i: (i, 0)),
                   pl.BlockSpec((64, 512), lambda i: (i, 0))),
        compiler_params=pltpu.CompilerParams(
            dimension_semantics=("parallel",), vmem_limit_bytes=_VMEM),
    )(xp3, _wcat(enc_se3_w),
      enc_se3_b.reshape(1, 128).astype(jnp.float32),
      _wcat(enc_sc01_w), enc_sc01_b.reshape(1, 256).astype(jnp.float32),
      _wcat(enc_sc12_w), enc_sc12_b.reshape(1, 512).astype(jnp.float32))

    mn2, mr2 = _norm_rows(memory_w2), memory_w2.astype(jnp.bfloat16)
    mn1, mr1 = _norm_rows(memory_w1), memory_w1.astype(jnp.bfloat16)
    mn0, mr0 = _norm_rows(memory_w0), memory_w0.astype(jnp.bfloat16)

    def f32b(v):
        return v.reshape(1, -1).astype(jnp.float32)

    cpar = pltpu.CompilerParams(
        dimension_semantics=("parallel", "arbitrary"), vmem_limit_bytes=_VMEM)
    bh = B // 2

    y2 = pl.pallas_call(
        functools.partial(_head2_body, bh=bh, T=T),
        out_shape=jax.ShapeDtypeStruct((B * 64, 1024), jnp.bfloat16),
        grid=(2, T),
        in_specs=[
            pl.BlockSpec((bh, 1, 64, 512), lambda i, t: (i, t, 0, 0)),
            pl.BlockSpec((1024, 512), lambda i, t: (0, t)),
            _bspec((1, 1024)),
            _bspec((512, 1024)), _bspec((1, 512)),
            _bspec((256, 512)), _bspec((256, 512)),
            _bspec((2048, 1024)), _bspec((1, 256)),
        ],
        out_specs=pl.BlockSpec((bh * 64, 1024), lambda i, t: (i, 0)),
        scratch_shapes=[pltpu.VMEM((bh * 64, 1024), jnp.float32)],
        compiler_params=cpar,
    )(x2r.reshape(B, T, 64, 512),
      enc_t201_w.reshape(1024, 2048).astype(jnp.bfloat16),
      f32b(enc_t201_b),
      enc_t212_w.reshape(512, 1024).astype(jnp.bfloat16), f32b(enc_t212_b),
      mn2, mr2, _convT_weight(emb21_w), f32b(emb21_b))

    y10, y11 = pl.pallas_call(
        functools.partial(_head1_body, bh=bh, T=T),
        out_shape=(jax.ShapeDtypeStruct((B * 256, 512), jnp.bfloat16),
                   jax.ShapeDtypeStruct((B * 256, 512), jnp.bfloat16)),
        grid=(2, T),
        in_specs=[
            pl.BlockSpec((bh, 1, 256, 256), lambda i, t: (i, t, 0, 0)),
            pl.BlockSpec((bh * 64, 1024), lambda i, t: (i, 0)),
            pl.BlockSpec((512, 256), lambda i, t: (0, t)),
            _bspec((1, 512)),
            _bspec((256, 512)), _bspec((1, 256)),
            _bspec((512, 256)), _bspec((512, 256)),
            _bspec((1024, 512)), _bspec((1, 128)),
            _bspec((1024, 512)), _bspec((1, 128)),
        ],
        out_specs=(pl.BlockSpec((bh * 256, 512), lambda i, t: (i, 0)),
                   pl.BlockSpec((bh * 256, 512), lambda i, t: (i, 0))),
        scratch_shapes=[pltpu.VMEM((bh * 256, 512), jnp.float32)],
        compiler_params=cpar,
    )(x1r.reshape(B, T, 256, 256), y2,
      enc_t101_w.reshape(512, 1024).astype(jnp.bfloat16), f32b(enc_t101_b),
      enc_t112_w.reshape(256, 512).astype(jnp.bfloat16), f32b(enc_t112_b),
      mn1, mr1, _convT_weight(emb10_w), f32b(emb10_b),
      _convT_weight(emb11_w), f32b(emb11_b))

    bq = B // 4
    out = pl.pallas_call(
        functools.partial(_head0_body, bh=bq, T=T),
        out_shape=jax.ShapeDtypeStruct((B, 128, 32, 32), jnp.float32),
        grid=(4, T),
        in_specs=[
            pl.BlockSpec((bq, 1, 1024, 128), lambda i, t: (i, t, 0, 0)),
            pl.BlockSpec((bq, 256, 512), lambda i, t: (i, 0, 0)),
            pl.BlockSpec((bq, 256, 512), lambda i, t: (i, 0, 0)),
            pl.BlockSpec((256, 128), lambda i, t: (0, t)),
            pl.BlockSpec((1, 256), lambda i, t: (0, 0)),
            pl.BlockSpec((128, 256), lambda i, t: (0, 0)),
            pl.BlockSpec((1, 128), lambda i, t: (0, 0)),
            pl.BlockSpec((1024, 128), lambda i, t: (0, 0)),
            pl.BlockSpec((1024, 128), lambda i, t: (0, 0)),
            pl.BlockSpec((384, 128), lambda i, t: (0, 0)),
            pl.BlockSpec((1, 128), lambda i, t: (0, 0)),
        ],
        out_specs=pl.BlockSpec((bq, 128, 32, 32), lambda i, t: (i, 0, 0, 0)),
        scratch_shapes=[pltpu.VMEM((bq * 1024, 256), jnp.float32)],
        compiler_params=cpar,
    )(x0r.reshape(B, T, 1024, 128),
      y10.reshape(B, 256, 512), y11.reshape(B, 256, 512),
      enc_t001_w.reshape(256, 512).astype(jnp.bfloat16), f32b(enc_t001_b),
      enc_t012_w.reshape(128, 256).astype(jnp.bfloat16), f32b(enc_t012_b),
      mn0, mr0, fusion_w.reshape(384, 128).astype(jnp.bfloat16),
      f32b(fusion_b))
    return out


# speedup vs baseline: 1.5510x; 1.5510x over previous
import functools

import jax
import jax.numpy as jnp
from jax.experimental import pallas as pl
from jax.experimental.pallas import tpu as pltpu

_VMEM = 64 * 1024 * 1024


def _act(y, kind):
    if kind == "relu":
        return jnp.maximum(y, 0.0)
    if kind == "elu":
        return jnp.where(y > 0, y, jnp.exp(jnp.minimum(y, 0.0)) - 1.0)
    return y


def _dotT(x, w):
    return jax.lax.dot_general(x, w, (((1,), (1,)), ((), ())),
                               preferred_element_type=jnp.float32)


def _pad_hw(y3):
    h, w, c = y3.shape
    zr = jnp.zeros((1, w, c), y3.dtype)
    y3 = jnp.concatenate([zr, y3, zr], axis=0)
    zc = jnp.zeros((h + 2, 1, c), y3.dtype)
    return jnp.concatenate([zc, y3, zc], axis=1)


def _phase_split(yp):
    H, W, c = yp.shape
    return (yp.reshape(H // 2, 2, W // 2, 2, c)
            .transpose(1, 3, 0, 2, 4))


def _tap_cat(xp, oh, ow):
    c = xp.shape[-1]
    parts = [xp[kh:kh + oh, kw:kw + ow, :].reshape(oh * ow, c)
             .astype(jnp.bfloat16)
             for kh in range(3) for kw in range(3)]
    return jnp.concatenate(parts, axis=1)


def _tap_cat_s2(ph, oh, ow):
    c = ph.shape[-1]
    parts = []
    for kh in range(3):
        for kw in range(3):
            xs = ph[kh % 2, kw % 2,
                    kh // 2:kh // 2 + oh, kw // 2:kw // 2 + ow, :]
            parts.append(xs.reshape(oh * ow, c).astype(jnp.bfloat16))
    return jnp.concatenate(parts, axis=1)


def _front_body(x_ref, w0_ref, b0_ref, w1_ref, b1_ref, w2_ref, b2_ref,
                o_ref, *, T):
    t = pl.program_id(1)
    d = x_ref[0, pl.ds(t + 1, 1)][0] - x_ref[0, pl.ds(t, 1)][0]
    db = d.astype(jnp.bfloat16)
    H2 = db.shape[0]
    Hq = H2 // 2
    k_ = jax.lax.broadcasted_iota(jnp.int32, (H2, H2), 0)
    j_ = jax.lax.broadcasted_iota(jnp.int32, (H2, H2), 1)
    sel = (j_ == (k_ // 2) + Hq * (k_ % 2)).astype(jnp.bfloat16)
    z = jax.lax.dot_general(
        jax.lax.dot_general(sel, db, (((0,), (0,)), ((), ())),
                            preferred_element_type=jnp.float32)
        .astype(jnp.bfloat16),
        sel, (((1,), (0,)), ((), ())),
        preferred_element_type=jnp.float32).astype(jnp.bfloat16)
    q = {}
    for ra in (0, 1):
        for cb in (0, 1):
            p = z[ra * Hq:(ra + 1) * Hq, cb * Hq:(cb + 1) * Hq]
            if ra:
                p = jnp.concatenate([jnp.zeros((1, Hq), p.dtype), p], axis=0)
            if cb:
                p = jnp.concatenate([jnp.zeros((p.shape[0], 1), p.dtype), p],
                                    axis=1)
            q[(ra, cb)] = p
    t9 = jnp.stack([q[((kh + 1) & 1, (kw + 1) & 1)]
                    [kh // 2:kh // 2 + 64, kw // 2:kw // 2 + 64]
                    for kh in range(3) for kw in range(3)], axis=-1)
    a0 = t9.reshape(4096, 9).astype(jnp.bfloat16)
    y0 = jnp.dot(a0, w0_ref[...], preferred_element_type=jnp.float32)
    y0 = _act(y0 + b0_ref[...], "elu")
    a1 = _tap_cat(_pad_hw(y0.reshape(64, 64, 64)), 64, 64)
    y1 = jnp.dot(a1, w1_ref[...], preferred_element_type=jnp.float32)
    y1 = _act(y1 + b1_ref[...], "elu")
    p1 = _phase_split(_pad_hw(y1.reshape(64, 64, 64)))
    a2 = _tap_cat_s2(p1, 32, 32)
    y2 = jnp.dot(a2, w2_ref[...], preferred_element_type=jnp.float32)
    y2 = _act(y2 + b2_ref[...], "elu")
    o_ref[...] = _pad_hw(y2.reshape(32, 32, 128))[None]


def _tail_body(x_ref, w3_ref, b3_ref, w4_ref, b4_ref, w5_ref, b5_ref,
               o0_ref, o1_ref, o2_ref):
    a3 = _tap_cat(x_ref[0], 32, 32)
    y3 = jnp.dot(a3, w3_ref[...], preferred_element_type=jnp.float32)
    y3 = _act(y3 + b3_ref[...], "elu")
    o0_ref[...] = y3.astype(jnp.bfloat16)[None, None]
    p3 = _phase_split(_pad_hw(y3.reshape(32, 32, 128)))
    a4 = _tap_cat_s2(p3, 16, 16)
    y4 = jnp.dot(a4, w4_ref[...], preferred_element_type=jnp.float32)
    y4 = jnp.maximum(y4 + b4_ref[...], 0.0)
    o1_ref[...] = y4.astype(jnp.bfloat16)[None, None]
    p4 = _phase_split(_pad_hw(y4.reshape(16, 16, 256)))
    a5 = _tap_cat_s2(p4, 8, 8)
    y5 = jnp.dot(a5, w5_ref[...], preferred_element_type=jnp.float32)
    o2_ref[...] = (jnp.maximum(y5 + b5_ref[...], 0.0)
                   .astype(jnp.bfloat16)[None, None])


def _wcat(w):
    OC, IC = w.shape[0], w.shape[1]
    return (w.transpose(2, 3, 1, 0).reshape(9 * IC, OC)
            .astype(jnp.bfloat16))


def _read(q, mem_n, mem_raw):
    qn = q * jax.lax.rsqrt(
        jnp.maximum(jnp.sum(q * q, axis=1, keepdims=True), 1e-24))
    s = _dotT(qn.astype(jnp.bfloat16), mem_n)
    s = jnp.exp(s - jnp.max(s, axis=1, keepdims=True))
    p = s / jnp.sum(s, axis=1, keepdims=True)
    return jnp.dot(p.astype(jnp.bfloat16), mem_raw,
                   preferred_element_type=jnp.float32)


def _patch4(x4d):
    b, h, w, C = x4d.shape
    xp = jnp.concatenate([x4d, jnp.zeros((b, 1, w, C), x4d.dtype)], axis=1)
    xp = jnp.concatenate([xp, jnp.zeros((b, h + 1, 1, C), x4d.dtype)], axis=2)
    cols = jnp.concatenate(
        [xp[:, di:di + h, dj:dj + w, :] for di in (0, 1) for dj in (0, 1)],
        axis=-1)
    return cols.reshape(b * h * w, 4 * C)


def _shuffle_rows(y, b, h, w, oc):
    return (y.reshape(b, h, w, 2, 2, oc).transpose(0, 1, 3, 2, 4, 5)
            .reshape(b, 2 * h, 2 * w, oc))


def _tile4(bias_ref):
    bv = bias_ref[...]
    return jnp.concatenate([bv, bv, bv, bv], axis=1)


def _finish_mlp(acc, b1_ref, w2_ref, b2_ref):
    h = jnp.maximum(acc + b1_ref[...], 0.0).astype(jnp.bfloat16)
    return jnp.maximum(_dotT(h, w2_ref[...]) + b2_ref[...], 0.0)


def _head2_body(x_ref, w1_ref, b1_ref, w2_ref, b2_ref, mn_ref, mr_ref,
                wc_ref, bc_ref, o_ref, acc_ref, *, bh, T):
    t = pl.program_id(1)

    @pl.when(t == 0)
    def _():
        acc_ref[...] = jnp.zeros_like(acc_ref)

    xr = x_ref[...].reshape(bh * 64, 512)
    acc_ref[...] += _dotT(xr, w1_ref[...])

    @pl.when(t == T - 1)
    def _():
        r2 = _finish_mlp(acc_ref[...], b1_ref, w2_ref, b2_ref)
        mf = _read(r2, mn_ref[...], mr_ref[...])
        cols = _patch4(mf.astype(jnp.bfloat16).reshape(bh, 8, 8, 512))
        y = jnp.dot(cols, wc_ref[...], preferred_element_type=jnp.float32)
        o_ref[...] = jnp.maximum(y + _tile4(bc_ref), 0.0).astype(o_ref.dtype)


def _head1_body(x_ref, y2_ref, w1_ref, b1_ref, w2_ref, b2_ref, mn_ref,
                mr_ref, w10_ref, b10_ref, w11_ref, b11_ref,
                o10_ref, o11_ref, acc_ref, *, bh, T):
    t = pl.program_id(1)

    @pl.when(t == 0)
    def _():
        acc_ref[...] = jnp.zeros_like(acc_ref)

    xr = x_ref[...].reshape(bh * 256, 256)
    acc_ref[...] += _dotT(xr, w1_ref[...])

    @pl.when(t == T - 1)
    def _():
        r1 = _finish_mlp(acc_ref[...], b1_ref, w2_ref, b2_ref)
        mf2 = _shuffle_rows(y2_ref[...], bh, 8, 8, 256)
        mq = (r1.astype(jnp.bfloat16)
              + mf2.reshape(bh * 256, 256)).astype(jnp.float32)
        mf = _read(mq, mn_ref[...], mr_ref[...])
        cols = _patch4(mf.astype(jnp.bfloat16).reshape(bh, 16, 16, 256))
        y = jnp.dot(cols, w10_ref[...], preferred_element_type=jnp.float32)
        o10_ref[...] = (jnp.maximum(y + _tile4(b10_ref), 0.0)
                        .astype(o10_ref.dtype).reshape(o10_ref.shape))
        cols2 = _patch4(mf2)
        y2 = jnp.dot(cols2, w11_ref[...], preferred_element_type=jnp.float32)
        o11_ref[...] = (jnp.maximum(y2 + _tile4(b11_ref), 0.0)
                        .astype(o11_ref.dtype).reshape(o11_ref.shape))


def _head0_body(x_ref, y10_ref, y11_ref, w1_ref, b1_ref, w2_ref, b2_ref,
                mn_ref, mr_ref, wf_ref, bf_ref, o_ref, acc_ref, *, bh, T):
    t = pl.program_id(1)

    @pl.when(t == 0)
    def _():
        acc_ref[...] = jnp.zeros_like(acc_ref)

    xr = x_ref[...].reshape(bh * 1024, 128)
    acc_ref[...] += _dotT(xr, w1_ref[...])

    @pl.when(t == T - 1)
    def _():
        r0 = _finish_mlp(acc_ref[...], b1_ref, w2_ref, b2_ref)
        mf1 = _shuffle_rows(y10_ref[...].reshape(bh * 256, 512),
                            bh, 16, 16, 128).reshape(bh * 1024, 128)
        up2 = _shuffle_rows(y11_ref[...].reshape(bh * 256, 512),
                            bh, 16, 16, 128).reshape(bh * 1024, 128)
        mq = (r0.astype(jnp.bfloat16) + mf1).astype(jnp.float32)
        mf0 = _read(mq, mn_ref[...], mr_ref[...])
        wf = wf_ref[...]
        f32 = jnp.float32
        y = (jnp.dot(mf0.astype(jnp.bfloat16), wf[:128],
                     preferred_element_type=f32)
             + jnp.dot(mf1, wf[128:256], preferred_element_type=f32)
             + jnp.dot(up2, wf[256:], preferred_element_type=f32))
        y = jnp.maximum(y + bf_ref[...], 0.0)
        yt = jnp.transpose(y.reshape(bh, 1024, 128), (0, 2, 1))
        o_ref[...] = yt.reshape(bh, 128, 32, 32)


def _norm_rows(m):
    mf = m.astype(jnp.float32)
    return (mf * jax.lax.rsqrt(
        jnp.maximum(jnp.sum(mf * mf, axis=1, keepdims=True), 1e-24))
            ).astype(jnp.bfloat16)


def _convT_weight(w):
    IC, OC = w.shape[0], w.shape[1]
    wp = jnp.pad(w, ((0, 0), (0, 0), (1, 1), (1, 1)))
    blocks = jnp.stack(
        [wp[:, :, 2 - 2 * di:4 - 2 * di, 2 - 2 * dj:4 - 2 * dj]
         for di in (0, 1) for dj in (0, 1)], axis=0)
    wc = blocks.transpose(0, 1, 3, 4, 2).reshape(4 * IC, 4 * OC)
    return wc.astype(jnp.bfloat16)


def _bspec(shape):
    n = len(shape)
    return pl.BlockSpec(shape, lambda i, t, _n=n: (0,) * _n)


def kernel(enc_se0_w, enc_se0_b, enc_se1_w, enc_se1_b, enc_se2_w, enc_se2_b,
           enc_se3_w, enc_se3_b, enc_sc01_w, enc_sc01_b, enc_sc12_w,
           enc_sc12_b, enc_t001_w, enc_t001_b, enc_t012_w, enc_t012_b,
           enc_t101_w, enc_t101_b, enc_t112_w, enc_t112_b, enc_t201_w,
           enc_t201_b, enc_t212_w, enc_t212_b, emb21_w, emb21_b, emb11_w,
           emb11_b, emb10_w, emb10_b, fusion_w, fusion_b, memory_w0,
           memory_w1, memory_w2, memory_x):
    B, L = memory_x.shape[0], memory_x.shape[1]
    T = L - 1
    H = memory_x.shape[3]
    N = B * T

    w0 = enc_se0_w.reshape(64, 9).T.astype(jnp.bfloat16)
    xp3 = pl.pallas_call(
        functools.partial(_front_body, T=T),
        out_shape=jax.ShapeDtypeStruct((N, 34, 34, 128), jnp.float32),
        grid=(B, T),
        in_specs=[
            pl.BlockSpec((1, L, H, H), lambda b, t: (b, 0, 0, 0)),
            pl.BlockSpec((9, 64), lambda b, t: (0, 0)),
            pl.BlockSpec((1, 64), lambda b, t: (0, 0)),
            pl.BlockSpec((576, 64), lambda b, t: (0, 0)),
            pl.BlockSpec((1, 64), lambda b, t: (0, 0)),
            pl.BlockSpec((576, 128), lambda b, t: (0, 0)),
            pl.BlockSpec((1, 128), lambda b, t: (0, 0)),
        ],
        out_specs=pl.BlockSpec((1, 34, 34, 128),
                               lambda b, t, _T=T: (b * _T + t, 0, 0, 0)),
        compiler_params=pltpu.CompilerParams(
            dimension_semantics=("parallel", "parallel"),
            vmem_limit_bytes=_VMEM),
    )(memory_x.reshape(B, L, H, H), w0,
      enc_se0_b.reshape(1, 64).astype(jnp.float32),
      _wcat(enc_se1_w), enc_se1_b.reshape(1, 64).astype(jnp.float32),
      _wcat(enc_se2_w), enc_se2_b.reshape(1, 128).astype(jnp.float32))

    x0r, x1r, x2r = pl.pallas_call(
        _tail_body,
        out_shape=(jax.ShapeDtypeStruct((B, T, 1024, 128), jnp.bfloat16),
                   jax.ShapeDtypeStruct((B, T, 256, 256), jnp.bfloat16),
                   jax.ShapeDtypeStruct((B, T, 64, 512), jnp.bfloat16)),
        grid=(N,),
        in_specs=[
            pl.BlockSpec((1, 34, 34, 128), lambda i: (i, 0, 0, 0)),
            pl.BlockSpec((1152, 128), lambda i: (0, 0)),
            pl.BlockSpec((1, 128), lambda i: (0, 0)),
            pl.BlockSpec((1152, 256), lambda i: (0, 0)),
            pl.BlockSpec((1, 256), lambda i: (0, 0)),
            pl.BlockSpec((2304, 512), lambda i: (0, 0)),
            pl.BlockSpec((1, 512), lambda i: (0, 0)),
        ],
        out_specs=(
            pl.BlockSpec((1, 1, 1024, 128),
                         lambda i, _T=T: (i // _T, i % _T, 0, 0)),
            pl.BlockSpec((1, 1, 256, 256),
                         lambda i, _T=T: (i // _T, i % _T, 0, 0)),
            pl.BlockSpec((1, 1, 64, 512),
                         lambda i, _T=T: (i // _T, i % _T, 0, 0))),
        compiler_params=pltpu.CompilerParams(
            dimension_semantics=("parallel",), vmem_limit_bytes=_VMEM),
    )(xp3, _wcat(enc_se3_w),
      enc_se3_b.reshape(1, 128).astype(jnp.float32),
      _wcat(enc_sc01_w), enc_sc01_b.reshape(1, 256).astype(jnp.float32),
      _wcat(enc_sc12_w), enc_sc12_b.reshape(1, 512).astype(jnp.float32))

    mn2, mr2 = _norm_rows(memory_w2), memory_w2.astype(jnp.bfloat16)
    mn1, mr1 = _norm_rows(memory_w1), memory_w1.astype(jnp.bfloat16)
    mn0, mr0 = _norm_rows(memory_w0), memory_w0.astype(jnp.bfloat16)

    def f32b(v):
        return v.reshape(1, -1).astype(jnp.float32)

    cpar = pltpu.CompilerParams(
        dimension_semantics=("parallel", "arbitrary"), vmem_limit_bytes=_VMEM)
    bh = B // 2

    y2 = pl.pallas_call(
        functools.partial(_head2_body, bh=bh, T=T),
        out_shape=jax.ShapeDtypeStruct((B * 64, 1024), jnp.bfloat16),
        grid=(2, T),
        in_specs=[
            pl.BlockSpec((bh, 1, 64, 512), lambda i, t: (i, t, 0, 0)),
            pl.BlockSpec((1024, 512), lambda i, t: (0, t)),
            _bspec((1, 1024)),
            _bspec((512, 1024)), _bspec((1, 512)),
            _bspec((256, 512)), _bspec((256, 512)),
            _bspec((2048, 1024)), _bspec((1, 256)),
        ],
        out_specs=pl.BlockSpec((bh * 64, 1024), lambda i, t: (i, 0)),
        scratch_shapes=[pltpu.VMEM((bh * 64, 1024), jnp.float32)],
        compiler_params=cpar,
    )(x2r,
      enc_t201_w.reshape(1024, 2048).astype(jnp.bfloat16),
      f32b(enc_t201_b),
      enc_t212_w.reshape(512, 1024).astype(jnp.bfloat16), f32b(enc_t212_b),
      mn2, mr2, _convT_weight(emb21_w), f32b(emb21_b))

    y10, y11 = pl.pallas_call(
        functools.partial(_head1_body, bh=bh, T=T),
        out_shape=(jax.ShapeDtypeStruct((B, 256, 512), jnp.bfloat16),
                   jax.ShapeDtypeStruct((B, 256, 512), jnp.bfloat16)),
        grid=(2, T),
        in_specs=[
            pl.BlockSpec((bh, 1, 256, 256), lambda i, t: (i, t, 0, 0)),
            pl.BlockSpec((bh * 64, 1024), lambda i, t: (i, 0)),
            pl.BlockSpec((512, 256), lambda i, t: (0, t)),
            _bspec((1, 512)),
            _bspec((256, 512)), _bspec((1, 256)),
            _bspec((512, 256)), _bspec((512, 256)),
            _bspec((1024, 512)), _bspec((1, 128)),
            _bspec((1024, 512)), _bspec((1, 128)),
        ],
        out_specs=(pl.BlockSpec((bh, 256, 512), lambda i, t: (i, 0, 0)),
                   pl.BlockSpec((bh, 256, 512), lambda i, t: (i, 0, 0))),
        scratch_shapes=[pltpu.VMEM((bh * 256, 512), jnp.float32)],
        compiler_params=cpar,
    )(x1r, y2,
      enc_t101_w.reshape(512, 1024).astype(jnp.bfloat16), f32b(enc_t101_b),
      enc_t112_w.reshape(256, 512).astype(jnp.bfloat16), f32b(enc_t112_b),
      mn1, mr1, _convT_weight(emb10_w), f32b(emb10_b),
      _convT_weight(emb11_w), f32b(emb11_b))

    bq = B // 4
    out = pl.pallas_call(
        functools.partial(_head0_body, bh=bq, T=T),
        out_shape=jax.ShapeDtypeStruct((B, 128, 32, 32), jnp.float32),
        grid=(4, T),
        in_specs=[
            pl.BlockSpec((bq, 1, 1024, 128), lambda i, t: (i, t, 0, 0)),
            pl.BlockSpec((bq, 256, 512), lambda i, t: (i, 0, 0)),
            pl.BlockSpec((bq, 256, 512), lambda i, t: (i, 0, 0)),
            pl.BlockSpec((256, 128), lambda i, t: (0, t)),
            pl.BlockSpec((1, 256), lambda i, t: (0, 0)),
            pl.BlockSpec((128, 256), lambda i, t: (0, 0)),
            pl.BlockSpec((1, 128), lambda i, t: (0, 0)),
            pl.BlockSpec((1024, 128), lambda i, t: (0, 0)),
            pl.BlockSpec((1024, 128), lambda i, t: (0, 0)),
            pl.BlockSpec((384, 128), lambda i, t: (0, 0)),
            pl.BlockSpec((1, 128), lambda i, t: (0, 0)),
        ],
        out_specs=pl.BlockSpec((bq, 128, 32, 32), lambda i, t: (i, 0, 0, 0)),
        scratch_shapes=[pltpu.VMEM((bq * 1024, 256), jnp.float32)],
        compiler_params=cpar,
    )(x0r, y10, y11,
      enc_t001_w.reshape(256, 512).astype(jnp.bfloat16), f32b(enc_t001_b),
      enc_t012_w.reshape(128, 256).astype(jnp.bfloat16), f32b(enc_t012_b),
      mn0, mr0, fusion_w.reshape(384, 128).astype(jnp.bfloat16),
      f32b(fusion_b))
    return out
